# Initial kernel scaffold; baseline (speedup 1.0000x reference)
import jax
import jax.numpy as jnp
from jax.experimental import pallas as pl


def kernel(image_adj_norm, image_adj, text_adj_norm, text_adj, ui_graph, iu_graph, image_table, text_table, image_feats_raw, text_feats_raw, W_img, b_img, W_txt, b_txt, bil_W, bil_b):
    raise NotImplementedError("write your pallas kernel here")


# speedup vs baseline: 1.0411x; 1.0411x over previous
import numpy as np
import jax
import jax.numpy as jnp
from jax.experimental import pallas as pl

_N_ITEMS = 4096
_N_USERS = 8192
_EMBED = 32


def _proj_body(img_t, txt_t, img_r, txt_r, wi, bi, wt, bt,
               o_xi, o_xt, o_ri, o_rt):
    o_xi[...] = jnp.dot(img_t[...], wi[...], preferred_element_type=jnp.float32) + bi[...]
    o_xt[...] = jnp.dot(txt_t[...], wt[...], preferred_element_type=jnp.float32) + bt[...]
    o_ri[...] = jnp.dot(img_r[...], wi[...], preferred_element_type=jnp.float32) + bi[...]
    o_rt[...] = jnp.dot(txt_r[...], wt[...], preferred_element_type=jnp.float32) + bt[...]


def _project(image_table, text_table, image_raw, text_raw, wi, bi, wt, bt):
    n, di = image_table.shape
    dt = text_table.shape[1]
    bm = 1024
    out = jax.ShapeDtypeStruct((n, _EMBED), jnp.float32)
    return pl.pallas_call(
        _proj_body,
        grid=(n // bm,),
        in_specs=[
            pl.BlockSpec((bm, di), lambda i: (i, 0)),
            pl.BlockSpec((bm, dt), lambda i: (i, 0)),
            pl.BlockSpec((bm, di), lambda i: (i, 0)),
            pl.BlockSpec((bm, dt), lambda i: (i, 0)),
            pl.BlockSpec((di, _EMBED), lambda i: (0, 0)),
            pl.BlockSpec((1, _EMBED), lambda i: (0, 0)),
            pl.BlockSpec((dt, _EMBED), lambda i: (0, 0)),
            pl.BlockSpec((1, _EMBED), lambda i: (0, 0)),
        ],
        out_specs=[pl.BlockSpec((bm, _EMBED), lambda i: (i, 0))] * 4,
        out_shape=[out] * 4,
    )(image_table, text_table, image_raw, text_raw, wi, bi, wt, bt)


def _prop_body(a_img, a_txt, x_img, x_txt, o_img, o_txt):
    o_img[...] = jnp.dot(a_img[...], x_img[...], preferred_element_type=jnp.float32)
    o_txt[...] = jnp.dot(a_txt[...], x_txt[...], preferred_element_type=jnp.float32)


def _propagate(image_adj, text_adj, x_img, x_txt, bm=256):
    n = image_adj.shape[0]
    out = jax.ShapeDtypeStruct((n, _EMBED), jnp.float32)
    return pl.pallas_call(
        _prop_body,
        grid=(n // bm,),
        in_specs=[
            pl.BlockSpec((bm, n), lambda i: (i, 0)),
            pl.BlockSpec((bm, n), lambda i: (i, 0)),
            pl.BlockSpec((n, _EMBED), lambda i: (0, 0)),
            pl.BlockSpec((n, _EMBED), lambda i: (0, 0)),
        ],
        out_specs=[pl.BlockSpec((bm, _EMBED), lambda i: (i, 0))] * 2,
        out_shape=[out] * 2,
    )(image_adj, text_adj, x_img, x_txt)


def _user_body(ui, h, o):
    o[...] = jnp.dot(ui[...], h[...], preferred_element_type=jnp.float32)


def _user(ui_graph, h_cat, bm=256):
    m, n = ui_graph.shape
    k = h_cat.shape[1]
    return pl.pallas_call(
        _user_body,
        grid=(m // bm,),
        in_specs=[
            pl.BlockSpec((bm, n), lambda i: (i, 0)),
            pl.BlockSpec((n, k), lambda i: (0, 0)),
        ],
        out_specs=pl.BlockSpec((bm, k), lambda i: (i, 0)),
        out_shape=jax.ShapeDtypeStruct((m, k), jnp.float32),
    )(ui_graph, h_cat)


def _ssl_body(hi, ht, ri, rt, w, b, o_ti, o_tt, o_fi, o_ft):
    colsum = jnp.sum((hi[...] + ht[...]) * 0.5, axis=0, keepdims=True)
    g = jax.nn.sigmoid(colsum)
    v = jnp.dot(g, w[...], preferred_element_type=jnp.float32)
    bb = b[0, 0]
    dn = (((1,), (1,)), ((), ()))
    o_ti[...] = jax.lax.dot_general(v, hi[...], dn, preferred_element_type=jnp.float32) + bb
    o_tt[...] = jax.lax.dot_general(v, ht[...], dn, preferred_element_type=jnp.float32) + bb
    o_fi[...] = jax.lax.dot_general(v, ri[...], dn, preferred_element_type=jnp.float32) + bb
    o_ft[...] = jax.lax.dot_general(v, rt[...], dn, preferred_element_type=jnp.float32) + bb


def _ssl(h_img, h_txt, r_img, r_txt, bil_w, bil_b):
    n = h_img.shape[0]
    out = jax.ShapeDtypeStruct((1, n), jnp.float32)
    full = lambda s: pl.BlockSpec(s, lambda: (0, 0))
    return pl.pallas_call(
        _ssl_body,
        in_specs=[full((n, _EMBED))] * 4 + [full((_EMBED, _EMBED)), full((1, 1))],
        out_specs=[full((1, n))] * 4,
        out_shape=[out] * 4,
    )(h_img, h_txt, r_img, r_txt, bil_w, bil_b)


def kernel(image_adj_norm, image_adj, text_adj_norm, text_adj, ui_graph, iu_graph,
           image_table, text_table, image_feats_raw, text_feats_raw,
           W_img, b_img, W_txt, b_txt, bil_W, bil_b):
    n_items = image_table.shape[0]
    rng = np.random.default_rng(0)
    idx_image = jnp.asarray(rng.permutation(n_items))
    idx_text = jnp.asarray(rng.permutation(n_items))

    bi = b_img.reshape(1, _EMBED)
    bt = b_txt.reshape(1, _EMBED)
    bw = bil_W.reshape(_EMBED, _EMBED)
    bb = bil_b.reshape(1, 1)

    x_img, x_txt, r_img, r_txt = _project(
        image_table, text_table, image_feats_raw, text_feats_raw,
        W_img, bi, W_txt, bt)

    h_img, h_txt = _propagate(image_adj, text_adj, x_img, x_txt)
    h_img, h_txt = _propagate(image_adj, text_adj, h_img, h_txt)

    user = _user(ui_graph, jnp.concatenate([h_img, h_txt], axis=1))
    user_img = user[:, :_EMBED]
    user_txt = user[:, _EMBED:]

    t_img, t_txt, f_img, f_txt = _ssl(h_img, h_txt, r_img, r_txt, bw, bb)
    ssl = jnp.concatenate(
        [t_img, t_txt, jnp.take(f_img, idx_image, axis=1),
         jnp.take(f_txt, idx_text, axis=1)], axis=1)
    return ssl, user_img, user_txt


# speedup vs baseline: 1.1471x; 1.1019x over previous
import numpy as np
import jax
import jax.numpy as jnp
from jax.experimental import pallas as pl
from jax.experimental.pallas import tpu as pltpu

_N = 4096
_M = 8192
_E = 32

_BP = 1024
_BL = 256
_BU = 256

_NP = _N // _BP
_NL = _N // _BL
_S_L1 = _NP
_S_L2 = _S_L1 + _NL
_S_SSL = _S_L2 + _NL
_STEPS = _S_SSL + 1


def _fused_body(ia, ta, itab, ttab, iraw, traw, wi, bi, wt, bt, bw, bb,
                h2c_o, ti_o, tt_o, fi_o, ft_o,
                x_i, x_t, r_i, r_t, h1_i, h1_t, h2_i, h2_t):
    s = pl.program_id(0)

    @pl.when(s < _S_L1)
    def _proj():
        rows = pl.ds(s * _BP, _BP)
        x_i[rows, :] = jnp.dot(itab[...], wi[...], preferred_element_type=jnp.float32) + bi[...]
        x_t[rows, :] = jnp.dot(ttab[...], wt[...], preferred_element_type=jnp.float32) + bt[...]
        r_i[rows, :] = jnp.dot(iraw[...], wi[...], preferred_element_type=jnp.float32) + bi[...]
        r_t[rows, :] = jnp.dot(traw[...], wt[...], preferred_element_type=jnp.float32) + bt[...]

    @pl.when((s >= _S_L1) & (s < _S_L2))
    def _layer1():
        rows = pl.ds((s - _S_L1) * _BL, _BL)
        h1_i[rows, :] = jnp.dot(ia[...], x_i[...], preferred_element_type=jnp.float32)
        h1_t[rows, :] = jnp.dot(ta[...], x_t[...], preferred_element_type=jnp.float32)

    @pl.when((s >= _S_L2) & (s < _S_SSL))
    def _layer2():
        rows = pl.ds((s - _S_L2) * _BL, _BL)
        h2_i[rows, :] = jnp.dot(ia[...], h1_i[...], preferred_element_type=jnp.float32)
        h2_t[rows, :] = jnp.dot(ta[...], h1_t[...], preferred_element_type=jnp.float32)

    @pl.when(s == _S_SSL)
    def _ssl():
        hi = h2_i[...]
        ht = h2_t[...]
        h2c_o[:, :_E] = hi
        h2c_o[:, _E:] = ht
        colsum = jnp.sum((hi + ht) * 0.5, axis=0, keepdims=True)
        g = jax.nn.sigmoid(colsum)
        v = jnp.dot(g, bw[...], preferred_element_type=jnp.float32)
        c = bb[0, 0]
        dn = (((1,), (1,)), ((), ()))
        ti_o[...] = jax.lax.dot_general(v, hi, dn, preferred_element_type=jnp.float32) + c
        tt_o[...] = jax.lax.dot_general(v, ht, dn, preferred_element_type=jnp.float32) + c
        fi_o[...] = jax.lax.dot_general(v, r_i[...], dn, preferred_element_type=jnp.float32) + c
        ft_o[...] = jax.lax.dot_general(v, r_t[...], dn, preferred_element_type=jnp.float32) + c


def _fused(image_adj, text_adj, image_table, text_table, image_raw, text_raw,
           wi, bi, wt, bt, bw, bb):
    di = image_table.shape[1]
    dt = text_table.shape[1]

    def adj_map(s):
        return (jnp.clip(jnp.where(s < _S_L2, s - _S_L1, s - _S_L2), 0, _NL - 1), 0)

    def tab_map(s):
        return (jnp.clip(s, 0, _NP - 1), 0)

    const2 = lambda s: (0, 0)
    f32 = jnp.float32
    outs = [
        jax.ShapeDtypeStruct((_N, 2 * _E), f32),
        jax.ShapeDtypeStruct((1, _N), f32),
        jax.ShapeDtypeStruct((1, _N), f32),
        jax.ShapeDtypeStruct((1, _N), f32),
        jax.ShapeDtypeStruct((1, _N), f32),
    ]
    return pl.pallas_call(
        _fused_body,
        grid=(_STEPS,),
        in_specs=[
            pl.BlockSpec((_BL, _N), adj_map),
            pl.BlockSpec((_BL, _N), adj_map),
            pl.BlockSpec((_BP, di), tab_map),
            pl.BlockSpec((_BP, dt), tab_map),
            pl.BlockSpec((_BP, di), tab_map),
            pl.BlockSpec((_BP, dt), tab_map),
            pl.BlockSpec((di, _E), const2),
            pl.BlockSpec((1, _E), const2),
            pl.BlockSpec((dt, _E), const2),
            pl.BlockSpec((1, _E), const2),
            pl.BlockSpec((_E, _E), const2),
            pl.BlockSpec((1, 1), const2),
        ],
        out_specs=[
            pl.BlockSpec((_N, 2 * _E), const2),
            pl.BlockSpec((1, _N), const2),
            pl.BlockSpec((1, _N), const2),
            pl.BlockSpec((1, _N), const2),
            pl.BlockSpec((1, _N), const2),
        ],
        out_shape=outs,
        scratch_shapes=[pltpu.VMEM((_N, _E), f32)] * 8,
    )(image_adj, text_adj, image_table, text_table, image_raw, text_raw,
      wi, bi, wt, bt, bw, bb)


def _user_body(ui, h, o):
    o[...] = jnp.dot(ui[...], h[...], preferred_element_type=jnp.float32)


def _user(ui_graph, h_cat):
    m, n = ui_graph.shape
    k = h_cat.shape[1]
    return pl.pallas_call(
        _user_body,
        grid=(m // _BU,),
        in_specs=[
            pl.BlockSpec((_BU, n), lambda i: (i, 0)),
            pl.BlockSpec((n, k), lambda i: (0, 0)),
        ],
        out_specs=pl.BlockSpec((_BU, k), lambda i: (i, 0)),
        out_shape=jax.ShapeDtypeStruct((m, k), jnp.float32),
    )(ui_graph, h_cat)


def kernel(image_adj_norm, image_adj, text_adj_norm, text_adj, ui_graph, iu_graph,
           image_table, text_table, image_feats_raw, text_feats_raw,
           W_img, b_img, W_txt, b_txt, bil_W, bil_b):
    n_items = image_table.shape[0]
    rng = np.random.default_rng(0)
    idx_image = jnp.asarray(rng.permutation(n_items))
    idx_text = jnp.asarray(rng.permutation(n_items))

    h2c, t_img, t_txt, f_img, f_txt = _fused(
        image_adj, text_adj, image_table, text_table,
        image_feats_raw, text_feats_raw,
        W_img, b_img.reshape(1, _E), W_txt, b_txt.reshape(1, _E),
        bil_W.reshape(_E, _E), bil_b.reshape(1, 1))

    user = _user(ui_graph, h2c)

    ssl = jnp.concatenate(
        [t_img, t_txt, jnp.take(f_img, idx_image, axis=1),
         jnp.take(f_txt, idx_text, axis=1)], axis=1)
    return ssl, user[:, :_E], user[:, _E:]


# speedup vs baseline: 1.1600x; 1.0112x over previous
import numpy as np
import jax
import jax.numpy as jnp
from jax.experimental import pallas as pl
from jax.experimental.pallas import tpu as pltpu

_N = 4096
_M = 8192
_E = 32

_BP = 256
_BL = 512
_BU = 512

_NP = _N // _BP
_NL = _N // _BL
_S_L1 = _NP
_S_L2 = _S_L1 + _NL
_S_SSL = _S_L2 + _NL
_STEPS = _S_SSL + 1


def _fused_body(ia, ta, itab, ttab, iraw, traw, wi, bi, wt, bt, bw, bb,
                h2c_o, ti_o, tt_o, fi_o, ft_o,
                x_i, x_t, r_i, r_t, h1_i, h1_t, h2_i, h2_t):
    s = pl.program_id(0)

    @pl.when(s < _S_L1)
    def _proj():
        rows = pl.ds(s * _BP, _BP)
        x_i[rows, :] = jnp.dot(itab[...], wi[...], preferred_element_type=jnp.float32) + bi[...]
        x_t[rows, :] = jnp.dot(ttab[...], wt[...], preferred_element_type=jnp.float32) + bt[...]
        r_i[rows, :] = jnp.dot(iraw[...], wi[...], preferred_element_type=jnp.float32) + bi[...]
        r_t[rows, :] = jnp.dot(traw[...], wt[...], preferred_element_type=jnp.float32) + bt[...]

    @pl.when((s >= _S_L1) & (s < _S_L2))
    def _layer1():
        rows = pl.ds((s - _S_L1) * _BL, _BL)
        h1_i[rows, :] = jnp.dot(ia[...], x_i[...], preferred_element_type=jnp.float32)
        h1_t[rows, :] = jnp.dot(ta[...], x_t[...], preferred_element_type=jnp.float32)

    @pl.when((s >= _S_L2) & (s < _S_SSL))
    def _layer2():
        rows = pl.ds((s - _S_L2) * _BL, _BL)
        h2_i[rows, :] = jnp.dot(ia[...], h1_i[...], preferred_element_type=jnp.float32)
        h2_t[rows, :] = jnp.dot(ta[...], h1_t[...], preferred_element_type=jnp.float32)

    @pl.when(s == _S_SSL)
    def _ssl():
        hi = h2_i[...]
        ht = h2_t[...]
        h2c_o[:, :_E] = hi
        h2c_o[:, _E:] = ht
        colsum = jnp.sum((hi + ht) * 0.5, axis=0, keepdims=True)
        g = jax.nn.sigmoid(colsum)
        v = jnp.dot(g, bw[...], preferred_element_type=jnp.float32)
        c = bb[0, 0]
        dn = (((1,), (1,)), ((), ()))
        ti_o[...] = jax.lax.dot_general(v, hi, dn, preferred_element_type=jnp.float32) + c
        tt_o[...] = jax.lax.dot_general(v, ht, dn, preferred_element_type=jnp.float32) + c
        fi_o[...] = jax.lax.dot_general(v, r_i[...], dn, preferred_element_type=jnp.float32) + c
        ft_o[...] = jax.lax.dot_general(v, r_t[...], dn, preferred_element_type=jnp.float32) + c


def _fused(image_adj, text_adj, image_table, text_table, image_raw, text_raw,
           wi, bi, wt, bt, bw, bb):
    di = image_table.shape[1]
    dt = text_table.shape[1]

    def adj_map(s):
        return (jnp.clip(jnp.where(s < _S_L2, s - _S_L1, s - _S_L2), 0, _NL - 1), 0)

    def tab_map(s):
        return (jnp.clip(s, 0, _NP - 1), 0)

    const2 = lambda s: (0, 0)
    f32 = jnp.float32
    outs = [
        jax.ShapeDtypeStruct((_N, 2 * _E), f32),
        jax.ShapeDtypeStruct((1, _N), f32),
        jax.ShapeDtypeStruct((1, _N), f32),
        jax.ShapeDtypeStruct((1, _N), f32),
        jax.ShapeDtypeStruct((1, _N), f32),
    ]
    return pl.pallas_call(
        _fused_body,
        grid=(_STEPS,),
        in_specs=[
            pl.BlockSpec((_BL, _N), adj_map),
            pl.BlockSpec((_BL, _N), adj_map),
            pl.BlockSpec((_BP, di), tab_map),
            pl.BlockSpec((_BP, dt), tab_map),
            pl.BlockSpec((_BP, di), tab_map),
            pl.BlockSpec((_BP, dt), tab_map),
            pl.BlockSpec((di, _E), const2),
            pl.BlockSpec((1, _E), const2),
            pl.BlockSpec((dt, _E), const2),
            pl.BlockSpec((1, _E), const2),
            pl.BlockSpec((_E, _E), const2),
            pl.BlockSpec((1, 1), const2),
        ],
        out_specs=[
            pl.BlockSpec((_N, 2 * _E), const2),
            pl.BlockSpec((1, _N), const2),
            pl.BlockSpec((1, _N), const2),
            pl.BlockSpec((1, _N), const2),
            pl.BlockSpec((1, _N), const2),
        ],
        out_shape=outs,
        scratch_shapes=[pltpu.VMEM((_N, _E), f32)] * 8,
    )(image_adj, text_adj, image_table, text_table, image_raw, text_raw,
      wi, bi, wt, bt, bw, bb)


def _user_body(ui, h, o_img, o_txt):
    res = jnp.dot(ui[...], h[...], preferred_element_type=jnp.float32)
    o_img[...] = res[:, :_E]
    o_txt[...] = res[:, _E:]


def _user(ui_graph, h_cat):
    m, n = ui_graph.shape
    k = h_cat.shape[1]
    return pl.pallas_call(
        _user_body,
        grid=(m // _BU,),
        in_specs=[
            pl.BlockSpec((_BU, n), lambda i: (i, 0)),
            pl.BlockSpec((n, k), lambda i: (0, 0)),
        ],
        out_specs=[pl.BlockSpec((_BU, _E), lambda i: (i, 0))] * 2,
        out_shape=[jax.ShapeDtypeStruct((m, _E), jnp.float32)] * 2,
    )(ui_graph, h_cat)


def kernel(image_adj_norm, image_adj, text_adj_norm, text_adj, ui_graph, iu_graph,
           image_table, text_table, image_feats_raw, text_feats_raw,
           W_img, b_img, W_txt, b_txt, bil_W, bil_b):
    n_items = image_table.shape[0]
    rng = np.random.default_rng(0)
    idx_image = jnp.asarray(rng.permutation(n_items))
    idx_text = jnp.asarray(rng.permutation(n_items))

    h2c, t_img, t_txt, f_img, f_txt = _fused(
        image_adj, text_adj, image_table, text_table,
        image_feats_raw, text_feats_raw,
        W_img, b_img.reshape(1, _E), W_txt, b_txt.reshape(1, _E),
        bil_W.reshape(_E, _E), bil_b.reshape(1, 1))

    user_img, user_txt = _user(ui_graph, h2c)

    ssl = jnp.concatenate(
        [t_img, t_txt, jnp.take(f_img, idx_image, axis=1),
         jnp.take(f_txt, idx_text, axis=1)], axis=1)
    return ssl, user_img, user_txt
